# baseline (speedup 1.0000x reference)
import functools

import jax
import jax.numpy as jnp
from jax import lax
from jax.experimental import pallas as pl
from jax.experimental.pallas import tpu as pltpu
from jax.experimental.pallas import tpu_sc as plsc

_NU = 5000
_NI = 5000
_N = _NU + _NI
_D = 128
_E = 320000
_B = 4096
_NC, _NS, _L = 2, 16, 16
_NW = _NC * _NS
_C = 128
_CHUNKS_W = 81
_TA = 39
_TB = 2 * _CHUNKS_W - _TA
_EPW = _CHUNKS_W * _C
_EPAD = _NW * _EPW
_NCH = _EPAD // _C
_RPT = 632
_PB = _B // _NW

_mesh = plsc.VectorSubcoreMesh(
    core_axis_name="c", subcore_axis_name="s", num_cores=_NC, num_subcores=_NS
)


def _worker_id():
    return lax.axis_index("s") * _NC + lax.axis_index("c")


def _bcast_lane(v16, lane):
    idx = jnp.full((_L,), lane, jnp.int32)
    return lax.gather(
        v16,
        idx[:, None],
        lax.GatherDimensionNumbers(
            offset_dims=(), collapsed_slice_dims=(0,), start_index_map=(0,)
        ),
        slice_sizes=(1,),
        mode=lax.GatherScatterMode.PROMISE_IN_BOUNDS,
    )


@functools.partial(
    pl.kernel,
    mesh=_mesh,
    out_type=jax.ShapeDtypeStruct((_NC, _N, _D), jnp.float32),
    scratch_types=(
        [pltpu.VMEM((_C, _D), jnp.float32) for _ in range(3)]
        + [pltpu.VMEM((2 * _C,), jnp.int32) for _ in range(3)]
        + [pltpu.VMEM((_C,), jnp.int32) for _ in range(3)]
        + [pltpu.VMEM((_C,), jnp.int32) for _ in range(3)]
        + [pltpu.VMEM((_C,), jnp.float32) for _ in range(3)]
        + [pltpu.VMEM_SHARED((_N, _D), jnp.float32)]
        + [pltpu.SemaphoreType.DMA for _ in range(9)]
    ),
)
def _layer(table, pkw, out,
           r0, r1, r2, f0, f1, f2, s0, s1, s2, d0, d1, d2, w0, w1, w2,
           acc, *sems):
    c = lax.axis_index("c")
    s = lax.axis_index("s")
    wid = _worker_id()
    rows = (r0, r1, r2)
    fb = (f0, f1, f2)
    srcv = (s0, s1, s2)
    dstv = (d0, d1, d2)
    wv = (w0, w1, w2)
    fsem = sems[0:3]
    gsem = sems[3:6]
    asem = sems[6:9]
    cbase = s * (_TA + _TB) + jnp.where(c == 0, 0, _TA)
    cnt = jnp.where(c == 0, _TA, _TB)

    def _zrow(r, _):
        for j in range(_D // _L):
            r0[r, pl.ds(j * _L, _L)] = jnp.zeros((_L,), jnp.float32)
        return 0

    lax.fori_loop(0, _C, _zrow, 0)
    for k in range(4):
        pltpu.sync_copy(
            r0.at[pl.ds(0, _C)], acc.at[pl.ds(s * _RPT + k * _C, _C)]
        )

    @pl.when(s < _NS - 1)
    def _():
        pltpu.sync_copy(
            r0.at[pl.ds(0, 120)], acc.at[pl.ds(s * _RPT + 512, 120)]
        )

    @pl.when(s == _NS - 1)
    def _():
        pltpu.sync_copy(
            r0.at[pl.ds(0, 8)], acc.at[pl.ds(s * _RPT + 512, 8)]
        )

    def _enq_f(t, j):
        pltpu.async_copy(
            pkw.at[pl.ds((cbase + t) * 2 * _C, 2 * _C)], fb[j], fsem[j]
        )

    def _wait_f(j):
        pltpu.make_async_copy(pkw.at[pl.ds(0, 2 * _C)], fb[j], fsem[j]).wait()

    def _wait_g(j):
        pltpu.make_async_copy(table.at[srcv[j]], rows[j], gsem[j]).wait()

    def _wait_a(j):
        pltpu.make_async_copy(rows[j], acc.at[dstv[j]], asem[j]).wait()

    def _enq_g(j):
        pltpu.async_copy(table.at[srcv[j]], rows[j], gsem[j])

    def _enq_a(j):
        pltpu.async_copy(rows[j], acc.at[dstv[j]], asem[j], add=True)

    def _unpack(j):
        def _ug(g, _):
            v = fb[j][pl.ds(g * _L, _L)]
            srcv[j][pl.ds(g * _L, _L)] = v & 0x3FFF
            dstv[j][pl.ds(g * _L, _L)] = lax.shift_right_logical(v, 14)
            q = fb[j][pl.ds(_C + g * _L, _L)]
            wv[j][pl.ds(g * _L, _L)] = q.astype(jnp.float32) * (2.0 ** -20)
            return 0

        lax.fori_loop(0, _C // _L, _ug, 0)

    def _scale(j):
        def _row(r, _):
            w16 = wv[j][pl.ds((r // _L) * _L, _L)]
            b = _bcast_lane(w16, r % _L)
            for q in range(_D // _L):
                rows[j][r, pl.ds(q * _L, _L)] = (
                    rows[j][r, pl.ds(q * _L, _L)] * b
                )
            return 0

        lax.fori_loop(0, _C, _row, 0)

    for i in range(3):
        _enq_f(i, i)
    _wait_f(0)
    _unpack(0)
    _enq_g(0)
    plsc.subcore_barrier()

    for t in range(3):
        j = t % 3
        jn = (t + 1) % 3
        _wait_g(j)
        _wait_f(jn)
        _unpack(jn)
        if t == 2:
            _wait_a(0)
        _enq_g(jn)
        _scale(j)
        _enq_a(j)
        _enq_f(t + 3, j)

    def _body(r, _):
        for j in range(3):
            t = r * 3 + j
            jn = (j + 1) % 3
            _wait_g(j)
            _wait_f(jn)
            _unpack(jn)
            _wait_a(jn)
            _enq_g(jn)
            _scale(j)
            _enq_a(j)
            _enq_f(t + 3, j)
        return 0

    lax.fori_loop(1, cnt // 3 - 1, _body, 0)

    for i in range(3):
        t = cnt - 3 + i
        j = i
        jn = (i + 1) % 3
        _wait_g(j)
        if i < 2:
            _wait_f(jn)
            _unpack(jn)
            _wait_a(jn)
            _enq_g(jn)
        _scale(j)
        _enq_a(j)

    for j in range(3):
        _wait_a(j)
    plsc.subcore_barrier()

    @pl.when(s < _NS - 1)
    def _():
        pltpu.sync_copy(
            acc.at[pl.ds(s * _RPT, _RPT)], out.at[c, pl.ds(s * _RPT, _RPT)]
        )

    @pl.when(s == _NS - 1)
    def _():
        pltpu.sync_copy(
            acc.at[pl.ds(s * _RPT, _N - 15 * _RPT)],
            out.at[c, pl.ds(s * _RPT, _N - 15 * _RPT)],
        )


def _combine_body(p0_ref, p1_ref, prev_ref, emb_ref, acc_ref):
    e = p0_ref[...] + p1_ref[...]
    emb_ref[...] = e
    acc_ref[...] = prev_ref[...] + e


def _combine(p0, p1, prev_acc):
    blk = 1000
    spec = pl.BlockSpec((blk, _D), lambda i: (i, 0))
    return pl.pallas_call(
        _combine_body,
        grid=(_N // blk,),
        in_specs=[spec, spec, spec],
        out_specs=[spec, spec],
        out_shape=[
            jax.ShapeDtypeStruct((_N, _D), jnp.float32),
            jax.ShapeDtypeStruct((_N, _D), jnp.float32),
        ],
    )(p0, p1, prev_acc)


def _light_body(p0_ref, p1_ref, prev_ref, light_ref):
    light_ref[...] = (
        prev_ref[...] + p0_ref[...] + p1_ref[...]
    ) * 0.25


def _light(p0, p1, prev_acc):
    blk = 1000
    spec = pl.BlockSpec((blk, _D), lambda i: (i, 0))
    return pl.pallas_call(
        _light_body,
        grid=(_N // blk,),
        in_specs=[spec, spec, spec],
        out_specs=spec,
        out_shape=jax.ShapeDtypeStruct((_N, _D), jnp.float32),
    )(p0, p1, prev_acc)


@functools.partial(
    pl.kernel,
    mesh=_mesh,
    out_type=(
        jax.ShapeDtypeStruct((_B, _D), jnp.float32),
        jax.ShapeDtypeStruct((_B, _D), jnp.float32),
        jax.ShapeDtypeStruct((_B, _D), jnp.float32),
        jax.ShapeDtypeStruct((_B, _D), jnp.float32),
    ),
    scratch_types=[
        pltpu.VMEM((_PB,), jnp.int32),
        pltpu.VMEM((_PB,), jnp.int32),
        pltpu.VMEM((_PB, _D), jnp.float32),
        pltpu.SemaphoreType.DMA,
    ],
)
def _gather_batch(user_emb, item_emb, light, users, items, ou, lu, oi, li,
                  idx_v, idx2_v, rows_v, sem):
    wid = _worker_id()
    base = wid * _PB
    hbm_slice = pl.ds(base, _PB)

    pltpu.sync_copy(users.at[hbm_slice], idx_v)
    pltpu.async_copy(user_emb.at[idx_v], rows_v, sem).wait()
    pltpu.sync_copy(rows_v, ou.at[hbm_slice])
    pltpu.async_copy(light.at[idx_v], rows_v, sem).wait()
    pltpu.sync_copy(rows_v, lu.at[hbm_slice])

    pltpu.sync_copy(items.at[hbm_slice], idx_v)
    pltpu.async_copy(item_emb.at[idx_v], rows_v, sem).wait()
    pltpu.sync_copy(rows_v, oi.at[hbm_slice])
    for k in range(_PB // _L):
        idx2_v[pl.ds(k * _L, _L)] = idx_v[pl.ds(k * _L, _L)] + _NU
    pltpu.async_copy(light.at[idx2_v], rows_v, sem).wait()
    pltpu.sync_copy(rows_v, li.at[hbm_slice])


def _gamma_body(ou_ref, lu_ref, oi_ref, li_ref, w1_ref, w2_ref, out_ref):
    ou = ou_ref[...]
    lu = lu_ref[...]
    oi = oi_ref[...]
    li = li_ref[...]
    w1 = w1_ref[...]
    w2 = w2_ref[...]

    z0u = jnp.sum(ou * w1[0, :_D][None, :] + lu * w1[0, _D:][None, :],
                  axis=1, keepdims=True)
    z1u = jnp.sum(ou * w1[1, :_D][None, :] + lu * w1[1, _D:][None, :],
                  axis=1, keepdims=True)
    a0u = 1.0 / (1.0 + jnp.exp(z1u - z0u))
    urow = ou * a0u + lu * (1.0 - a0u)

    z0i = jnp.sum(oi * w2[0, :_D][None, :] + li * w2[0, _D:][None, :],
                  axis=1, keepdims=True)
    z1i = jnp.sum(oi * w2[1, :_D][None, :] + li * w2[1, _D:][None, :],
                  axis=1, keepdims=True)
    a0i = 1.0 / (1.0 + jnp.exp(z1i - z0i))
    irow = oi * a0i + li * (1.0 - a0i)

    out_ref[...] = jnp.sum(urow * irow, axis=1)


def _gamma(ou, lu, oi, li, att1_t, att2_t):
    blk = 512
    spec = pl.BlockSpec((blk, _D), lambda i: (i, 0))
    wspec = pl.BlockSpec((2, 2 * _D), lambda i: (0, 0))
    return pl.pallas_call(
        _gamma_body,
        grid=(_B // blk,),
        in_specs=[spec, spec, spec, spec, wspec, wspec],
        out_specs=pl.BlockSpec((blk,), lambda i: (i,)),
        out_shape=jax.ShapeDtypeStruct((_B,), jnp.float32),
    )(ou, lu, oi, li, att1_t, att2_t)


def kernel(user_emb, item_emb, att_exp1, att_exp2, edge_weight, edge_index,
           users, items):
    emb0 = jnp.concatenate([user_emb, item_emb], axis=0)
    src = jnp.pad(edge_index[0], (0, _EPAD - _E))
    dst = jnp.pad(edge_index[1], (0, _EPAD - _E))
    wfix = (jnp.pad(edge_weight, (0, _EPAD - _E)) * 1048576.0).astype(
        jnp.int32
    )
    packed = jnp.left_shift(dst, 14) | src
    pkw = jnp.stack(
        [packed.reshape(_NCH, _C), wfix.reshape(_NCH, _C)], axis=1
    ).reshape(-1)

    p = _layer(emb0, pkw)
    emb1, acc1 = _combine(p[0], p[1], emb0)
    q = _layer(emb1, pkw)
    emb2, acc2 = _combine(q[0], q[1], acc1)
    r = _layer(emb2, pkw)
    light = _light(r[0], r[1], acc2)

    ou, lu, oi, li = _gather_batch(user_emb, item_emb, light, users, items)
    return _gamma(ou, lu, oi, li, att_exp1.T, att_exp2.T)


# speedup vs baseline: 1.2318x; 1.2318x over previous
import functools

import jax
import jax.numpy as jnp
from jax import lax
from jax.experimental import pallas as pl
from jax.experimental.pallas import tpu as pltpu
from jax.experimental.pallas import tpu_sc as plsc

_NU = 5000
_NI = 5000
_N = _NU + _NI
_D = 128
_E = 320000
_B = 4096
_NC, _NS, _L = 2, 16, 16
_NW = _NC * _NS
_C = 128
_CHUNKS_W = 81
_TA = 123
_TB = 2 * _CHUNKS_W - _TA
_EPW = _CHUNKS_W * _C
_EPAD = _NW * _EPW
_NCH = _EPAD // _C
_RPT = 632
_PB = _B // _NW

_mesh = plsc.VectorSubcoreMesh(
    core_axis_name="c", subcore_axis_name="s", num_cores=_NC, num_subcores=_NS
)


def _worker_id():
    return lax.axis_index("s") * _NC + lax.axis_index("c")


def _bcast_lane(v16, lane):
    idx = jnp.full((_L,), lane, jnp.int32)
    return lax.gather(
        v16,
        idx[:, None],
        lax.GatherDimensionNumbers(
            offset_dims=(), collapsed_slice_dims=(0,), start_index_map=(0,)
        ),
        slice_sizes=(1,),
        mode=lax.GatherScatterMode.PROMISE_IN_BOUNDS,
    )


@functools.partial(
    pl.kernel,
    mesh=_mesh,
    out_type=jax.ShapeDtypeStruct((_NC, _N, _D), jnp.float32),
    scratch_types=(
        [pltpu.VMEM((_C, _D), jnp.float32) for _ in range(3)]
        + [pltpu.VMEM((2 * _C,), jnp.int32) for _ in range(3)]
        + [pltpu.VMEM((_C,), jnp.int32) for _ in range(3)]
        + [pltpu.VMEM((_C,), jnp.int32) for _ in range(3)]
        + [pltpu.VMEM((_C,), jnp.float32) for _ in range(3)]
        + [pltpu.VMEM_SHARED((_N, _D), jnp.float32)]
        + [pltpu.SemaphoreType.DMA for _ in range(9)]
    ),
)
def _layer(table, pkw, out,
           r0, r1, r2, f0, f1, f2, s0, s1, s2, d0, d1, d2, w0, w1, w2,
           acc, *sems):
    c = lax.axis_index("c")
    s = lax.axis_index("s")
    wid = _worker_id()
    rows = (r0, r1, r2)
    fb = (f0, f1, f2)
    srcv = (s0, s1, s2)
    dstv = (d0, d1, d2)
    wv = (w0, w1, w2)
    fsem = sems[0:3]
    gsem = sems[3:6]
    asem = sems[6:9]
    cbase = s * (_TA + _TB) + jnp.where(c == 0, 0, _TA)
    cnt = jnp.where(c == 0, _TA, _TB)

    def _zrow(r, _):
        for j in range(_D // _L):
            r0[r, pl.ds(j * _L, _L)] = jnp.zeros((_L,), jnp.float32)
        return 0

    lax.fori_loop(0, _C, _zrow, 0)
    for k in range(4):
        pltpu.sync_copy(
            r0.at[pl.ds(0, _C)], acc.at[pl.ds(s * _RPT + k * _C, _C)]
        )

    @pl.when(s < _NS - 1)
    def _():
        pltpu.sync_copy(
            r0.at[pl.ds(0, 120)], acc.at[pl.ds(s * _RPT + 512, 120)]
        )

    @pl.when(s == _NS - 1)
    def _():
        pltpu.sync_copy(
            r0.at[pl.ds(0, 8)], acc.at[pl.ds(s * _RPT + 512, 8)]
        )

    def _enq_f(t, j):
        pltpu.async_copy(
            pkw.at[pl.ds((cbase + t) * 2 * _C, 2 * _C)], fb[j], fsem[j]
        )

    def _wait_f(j):
        pltpu.make_async_copy(pkw.at[pl.ds(0, 2 * _C)], fb[j], fsem[j]).wait()

    def _wait_g(j):
        pltpu.make_async_copy(table.at[srcv[j]], rows[j], gsem[j]).wait()

    def _wait_a(j):
        pltpu.make_async_copy(rows[j], acc.at[dstv[j]], asem[j]).wait()

    def _enq_g(j):
        pltpu.async_copy(table.at[srcv[j]], rows[j], gsem[j])

    def _enq_a(j):
        pltpu.async_copy(rows[j], acc.at[dstv[j]], asem[j], add=True)

    def _unpack(j):
        def _ug(g, _):
            v = fb[j][pl.ds(g * _L, _L)]
            srcv[j][pl.ds(g * _L, _L)] = v & 0x3FFF
            dstv[j][pl.ds(g * _L, _L)] = lax.shift_right_logical(v, 14)
            q = fb[j][pl.ds(_C + g * _L, _L)]
            wv[j][pl.ds(g * _L, _L)] = q.astype(jnp.float32) * (2.0 ** -20)
            return 0

        lax.fori_loop(0, _C // _L, _ug, 0)

    def _scale(j):
        def _row(r, _):
            w16 = wv[j][pl.ds((r // _L) * _L, _L)]
            b = _bcast_lane(w16, r % _L)
            for q in range(_D // _L):
                rows[j][r, pl.ds(q * _L, _L)] = (
                    rows[j][r, pl.ds(q * _L, _L)] * b
                )
            return 0

        lax.fori_loop(0, _C, _row, 0)

    for i in range(3):
        _enq_f(i, i)
    _wait_f(0)
    _unpack(0)
    _enq_g(0)
    plsc.subcore_barrier()

    for t in range(3):
        j = t % 3
        jn = (t + 1) % 3
        _wait_g(j)
        _wait_f(jn)
        _unpack(jn)
        if t == 2:
            _wait_a(0)
        _enq_g(jn)
        _scale(j)
        _enq_a(j)
        _enq_f(t + 3, j)

    def _body(r, _):
        for j in range(3):
            t = r * 3 + j
            jn = (j + 1) % 3
            _wait_g(j)
            _wait_f(jn)
            _unpack(jn)
            _wait_a(jn)
            _enq_g(jn)
            _scale(j)
            _enq_a(j)
            _enq_f(t + 3, j)
        return 0

    lax.fori_loop(1, cnt // 3 - 1, _body, 0)

    for i in range(3):
        t = cnt - 3 + i
        j = i
        jn = (i + 1) % 3
        _wait_g(j)
        if i < 2:
            _wait_f(jn)
            _unpack(jn)
            _wait_a(jn)
            _enq_g(jn)
        _scale(j)
        _enq_a(j)

    for j in range(3):
        _wait_a(j)
    plsc.subcore_barrier()

    @pl.when(s < _NS - 1)
    def _():
        pltpu.sync_copy(
            acc.at[pl.ds(s * _RPT, _RPT)], out.at[c, pl.ds(s * _RPT, _RPT)]
        )

    @pl.when(s == _NS - 1)
    def _():
        pltpu.sync_copy(
            acc.at[pl.ds(s * _RPT, _N - 15 * _RPT)],
            out.at[c, pl.ds(s * _RPT, _N - 15 * _RPT)],
        )


def _combine_body(p0_ref, p1_ref, prev_ref, emb_ref, acc_ref):
    e = p0_ref[...] + p1_ref[...]
    emb_ref[...] = e
    acc_ref[...] = prev_ref[...] + e


def _combine(p0, p1, prev_acc):
    blk = 1000
    spec = pl.BlockSpec((blk, _D), lambda i: (i, 0))
    return pl.pallas_call(
        _combine_body,
        grid=(_N // blk,),
        in_specs=[spec, spec, spec],
        out_specs=[spec, spec],
        out_shape=[
            jax.ShapeDtypeStruct((_N, _D), jnp.float32),
            jax.ShapeDtypeStruct((_N, _D), jnp.float32),
        ],
    )(p0, p1, prev_acc)


def _light_body(p0_ref, p1_ref, prev_ref, light_ref):
    light_ref[...] = (
        prev_ref[...] + p0_ref[...] + p1_ref[...]
    ) * 0.25


def _light(p0, p1, prev_acc):
    blk = 1000
    spec = pl.BlockSpec((blk, _D), lambda i: (i, 0))
    return pl.pallas_call(
        _light_body,
        grid=(_N // blk,),
        in_specs=[spec, spec, spec],
        out_specs=spec,
        out_shape=jax.ShapeDtypeStruct((_N, _D), jnp.float32),
    )(p0, p1, prev_acc)


@functools.partial(
    pl.kernel,
    mesh=_mesh,
    out_type=(
        jax.ShapeDtypeStruct((_B, _D), jnp.float32),
        jax.ShapeDtypeStruct((_B, _D), jnp.float32),
        jax.ShapeDtypeStruct((_B, _D), jnp.float32),
        jax.ShapeDtypeStruct((_B, _D), jnp.float32),
    ),
    scratch_types=[
        pltpu.VMEM((_PB,), jnp.int32),
        pltpu.VMEM((_PB,), jnp.int32),
        pltpu.VMEM((_PB, _D), jnp.float32),
        pltpu.SemaphoreType.DMA,
    ],
)
def _gather_batch(user_emb, item_emb, light, users, items, ou, lu, oi, li,
                  idx_v, idx2_v, rows_v, sem):
    wid = _worker_id()
    base = wid * _PB
    hbm_slice = pl.ds(base, _PB)

    pltpu.sync_copy(users.at[hbm_slice], idx_v)
    pltpu.async_copy(user_emb.at[idx_v], rows_v, sem).wait()
    pltpu.sync_copy(rows_v, ou.at[hbm_slice])
    pltpu.async_copy(light.at[idx_v], rows_v, sem).wait()
    pltpu.sync_copy(rows_v, lu.at[hbm_slice])

    pltpu.sync_copy(items.at[hbm_slice], idx_v)
    pltpu.async_copy(item_emb.at[idx_v], rows_v, sem).wait()
    pltpu.sync_copy(rows_v, oi.at[hbm_slice])
    for k in range(_PB // _L):
        idx2_v[pl.ds(k * _L, _L)] = idx_v[pl.ds(k * _L, _L)] + _NU
    pltpu.async_copy(light.at[idx2_v], rows_v, sem).wait()
    pltpu.sync_copy(rows_v, li.at[hbm_slice])


def _gamma_body(ou_ref, lu_ref, oi_ref, li_ref, w1_ref, w2_ref, out_ref):
    ou = ou_ref[...]
    lu = lu_ref[...]
    oi = oi_ref[...]
    li = li_ref[...]
    w1 = w1_ref[...]
    w2 = w2_ref[...]

    z0u = jnp.sum(ou * w1[0, :_D][None, :] + lu * w1[0, _D:][None, :],
                  axis=1, keepdims=True)
    z1u = jnp.sum(ou * w1[1, :_D][None, :] + lu * w1[1, _D:][None, :],
                  axis=1, keepdims=True)
    a0u = 1.0 / (1.0 + jnp.exp(z1u - z0u))
    urow = ou * a0u + lu * (1.0 - a0u)

    z0i = jnp.sum(oi * w2[0, :_D][None, :] + li * w2[0, _D:][None, :],
                  axis=1, keepdims=True)
    z1i = jnp.sum(oi * w2[1, :_D][None, :] + li * w2[1, _D:][None, :],
                  axis=1, keepdims=True)
    a0i = 1.0 / (1.0 + jnp.exp(z1i - z0i))
    irow = oi * a0i + li * (1.0 - a0i)

    out_ref[...] = jnp.sum(urow * irow, axis=1)


def _gamma(ou, lu, oi, li, att1_t, att2_t):
    blk = 512
    spec = pl.BlockSpec((blk, _D), lambda i: (i, 0))
    wspec = pl.BlockSpec((2, 2 * _D), lambda i: (0, 0))
    return pl.pallas_call(
        _gamma_body,
        grid=(_B // blk,),
        in_specs=[spec, spec, spec, spec, wspec, wspec],
        out_specs=pl.BlockSpec((blk,), lambda i: (i,)),
        out_shape=jax.ShapeDtypeStruct((_B,), jnp.float32),
    )(ou, lu, oi, li, att1_t, att2_t)


def kernel(user_emb, item_emb, att_exp1, att_exp2, edge_weight, edge_index,
           users, items):
    emb0 = jnp.concatenate([user_emb, item_emb], axis=0)
    src = jnp.pad(edge_index[0], (0, _EPAD - _E))
    dst = jnp.pad(edge_index[1], (0, _EPAD - _E))
    wfix = (jnp.pad(edge_weight, (0, _EPAD - _E)) * 1048576.0).astype(
        jnp.int32
    )
    packed = jnp.left_shift(dst, 14) | src
    pkw = jnp.stack(
        [packed.reshape(_NCH, _C), wfix.reshape(_NCH, _C)], axis=1
    ).reshape(-1)

    p = _layer(emb0, pkw)
    emb1, acc1 = _combine(p[0], p[1], emb0)
    q = _layer(emb1, pkw)
    emb2, acc2 = _combine(q[0], q[1], acc1)
    r = _layer(emb2, pkw)
    light = _light(r[0], r[1], acc2)

    ou, lu, oi, li = _gather_batch(user_emb, item_emb, light, users, items)
    return _gamma(ou, lu, oi, li, att_exp1.T, att_exp2.T)


# speedup vs baseline: 1.3898x; 1.1283x over previous
import functools

import jax
import jax.numpy as jnp
from jax import lax
from jax.experimental import pallas as pl
from jax.experimental.pallas import tpu as pltpu
from jax.experimental.pallas import tpu_sc as plsc

_NU = 5000
_NI = 5000
_N = _NU + _NI
_D = 128
_E = 320000
_B = 4096
_NC, _NS, _L = 2, 16, 16
_NW = _NC * _NS
_C = 128
_CHUNKS_W = 79
_EPW = _CHUNKS_W * _C
_EPAD = _NW * _EPW
_RPT = 632
_PB = _B // _NW

_mesh = plsc.VectorSubcoreMesh(
    core_axis_name="c", subcore_axis_name="s", num_cores=_NC, num_subcores=_NS
)


def _worker_id():
    return lax.axis_index("s") * _NC + lax.axis_index("c")


def _bcast_lane(v16, lane):
    idx = jnp.full((_L,), lane, jnp.int32)
    return lax.gather(
        v16,
        idx[:, None],
        lax.GatherDimensionNumbers(
            offset_dims=(), collapsed_slice_dims=(0,), start_index_map=(0,)
        ),
        slice_sizes=(1,),
        mode=lax.GatherScatterMode.PROMISE_IN_BOUNDS,
    )


@functools.partial(
    pl.kernel,
    mesh=_mesh,
    out_type=jax.ShapeDtypeStruct((_NC, _N, _D), jnp.float32),
    scratch_types=[
        pltpu.VMEM((_C,), jnp.int32),
        pltpu.VMEM((_C,), jnp.int32),
        pltpu.VMEM((_C,), jnp.float32),
        pltpu.VMEM((_C, _D), jnp.float32),
        pltpu.VMEM_SHARED((_N, _D), jnp.float32),
        pltpu.SemaphoreType.DMA,
    ],
)
def _layer(table, src, dst, w, out, src_v, dst_v, w_v, rows_v, acc, sem):
    c = lax.axis_index("c")
    s = lax.axis_index("s")
    wid = _worker_id()

    def _zrow(r, _):
        for j in range(_D // _L):
            rows_v[r, pl.ds(j * _L, _L)] = jnp.zeros((_L,), jnp.float32)
        return 0

    lax.fori_loop(0, _C, _zrow, 0)
    for k in range(4):
        pltpu.sync_copy(
            rows_v.at[pl.ds(0, _C)],
            acc.at[pl.ds(s * _RPT + k * _C, _C)],
        )

    @pl.when(s < _NS - 1)
    def _():
        pltpu.sync_copy(
            rows_v.at[pl.ds(0, 120)], acc.at[pl.ds(s * _RPT + 512, 120)]
        )

    @pl.when(s == _NS - 1)
    def _():
        pltpu.sync_copy(
            rows_v.at[pl.ds(0, 8)], acc.at[pl.ds(s * _RPT + 512, 8)]
        )

    plsc.subcore_barrier()

    def _chunk(t, _):
        off = wid * _EPW + t * _C
        pltpu.sync_copy(src.at[pl.ds(off, _C)], src_v)
        pltpu.sync_copy(dst.at[pl.ds(off, _C)], dst_v)
        pltpu.sync_copy(w.at[pl.ds(off, _C)], w_v)
        pltpu.async_copy(table.at[src_v], rows_v, sem).wait()

        def _scale(r, _):
            w16 = w_v[pl.ds((r // _L) * _L, _L)]
            b = _bcast_lane(w16, r % _L)
            for j in range(_D // _L):
                rows_v[r, pl.ds(j * _L, _L)] = (
                    rows_v[r, pl.ds(j * _L, _L)] * b
                )
            return 0

        lax.fori_loop(0, _C, _scale, 0)
        pltpu.sync_copy(rows_v, acc.at[dst_v], add=True)
        return 0

    lax.fori_loop(0, _CHUNKS_W, _chunk, 0)
    plsc.subcore_barrier()

    @pl.when(s < _NS - 1)
    def _():
        pltpu.sync_copy(
            acc.at[pl.ds(s * _RPT, _RPT)], out.at[c, pl.ds(s * _RPT, _RPT)]
        )

    @pl.when(s == _NS - 1)
    def _():
        pltpu.sync_copy(
            acc.at[pl.ds(s * _RPT, _N - 15 * _RPT)],
            out.at[c, pl.ds(s * _RPT, _N - 15 * _RPT)],
        )


def _combine_body(p0_ref, p1_ref, prev_ref, emb_ref, acc_ref):
    e = p0_ref[...] + p1_ref[...]
    emb_ref[...] = e
    acc_ref[...] = prev_ref[...] + e


def _combine(p0, p1, prev_acc):
    blk = 1000
    spec = pl.BlockSpec((blk, _D), lambda i: (i, 0))
    return pl.pallas_call(
        _combine_body,
        grid=(_N // blk,),
        in_specs=[spec, spec, spec],
        out_specs=[spec, spec],
        out_shape=[
            jax.ShapeDtypeStruct((_N, _D), jnp.float32),
            jax.ShapeDtypeStruct((_N, _D), jnp.float32),
        ],
    )(p0, p1, prev_acc)


def _light_body(p0_ref, p1_ref, prev_ref, light_ref):
    light_ref[...] = (
        prev_ref[...] + p0_ref[...] + p1_ref[...]
    ) * 0.25


def _light(p0, p1, prev_acc):
    blk = 1000
    spec = pl.BlockSpec((blk, _D), lambda i: (i, 0))
    return pl.pallas_call(
        _light_body,
        grid=(_N // blk,),
        in_specs=[spec, spec, spec],
        out_specs=spec,
        out_shape=jax.ShapeDtypeStruct((_N, _D), jnp.float32),
    )(p0, p1, prev_acc)


@functools.partial(
    pl.kernel,
    mesh=_mesh,
    out_type=(
        jax.ShapeDtypeStruct((_B, _D), jnp.float32),
        jax.ShapeDtypeStruct((_B, _D), jnp.float32),
        jax.ShapeDtypeStruct((_B, _D), jnp.float32),
        jax.ShapeDtypeStruct((_B, _D), jnp.float32),
    ),
    scratch_types=[
        pltpu.VMEM((_PB,), jnp.int32),
        pltpu.VMEM((_PB,), jnp.int32),
        pltpu.VMEM((_PB, _D), jnp.float32),
        pltpu.SemaphoreType.DMA,
    ],
)
def _gather_batch(user_emb, item_emb, light, users, items, ou, lu, oi, li,
                  idx_v, idx2_v, rows_v, sem):
    wid = _worker_id()
    base = wid * _PB
    hbm_slice = pl.ds(base, _PB)

    pltpu.sync_copy(users.at[hbm_slice], idx_v)
    pltpu.async_copy(user_emb.at[idx_v], rows_v, sem).wait()
    pltpu.sync_copy(rows_v, ou.at[hbm_slice])
    pltpu.async_copy(light.at[idx_v], rows_v, sem).wait()
    pltpu.sync_copy(rows_v, lu.at[hbm_slice])

    pltpu.sync_copy(items.at[hbm_slice], idx_v)
    pltpu.async_copy(item_emb.at[idx_v], rows_v, sem).wait()
    pltpu.sync_copy(rows_v, oi.at[hbm_slice])
    for k in range(_PB // _L):
        idx2_v[pl.ds(k * _L, _L)] = idx_v[pl.ds(k * _L, _L)] + _NU
    pltpu.async_copy(light.at[idx2_v], rows_v, sem).wait()
    pltpu.sync_copy(rows_v, li.at[hbm_slice])


def _gamma_body(ou_ref, lu_ref, oi_ref, li_ref, w1_ref, w2_ref, out_ref):
    ou = ou_ref[...]
    lu = lu_ref[...]
    oi = oi_ref[...]
    li = li_ref[...]
    w1 = w1_ref[...]
    w2 = w2_ref[...]

    z0u = jnp.sum(ou * w1[0, :_D][None, :] + lu * w1[0, _D:][None, :],
                  axis=1, keepdims=True)
    z1u = jnp.sum(ou * w1[1, :_D][None, :] + lu * w1[1, _D:][None, :],
                  axis=1, keepdims=True)
    a0u = 1.0 / (1.0 + jnp.exp(z1u - z0u))
    urow = ou * a0u + lu * (1.0 - a0u)

    z0i = jnp.sum(oi * w2[0, :_D][None, :] + li * w2[0, _D:][None, :],
                  axis=1, keepdims=True)
    z1i = jnp.sum(oi * w2[1, :_D][None, :] + li * w2[1, _D:][None, :],
                  axis=1, keepdims=True)
    a0i = 1.0 / (1.0 + jnp.exp(z1i - z0i))
    irow = oi * a0i + li * (1.0 - a0i)

    out_ref[...] = jnp.sum(urow * irow, axis=1)


def _gamma(ou, lu, oi, li, att1_t, att2_t):
    blk = 512
    spec = pl.BlockSpec((blk, _D), lambda i: (i, 0))
    wspec = pl.BlockSpec((2, 2 * _D), lambda i: (0, 0))
    return pl.pallas_call(
        _gamma_body,
        grid=(_B // blk,),
        in_specs=[spec, spec, spec, spec, wspec, wspec],
        out_specs=pl.BlockSpec((blk,), lambda i: (i,)),
        out_shape=jax.ShapeDtypeStruct((_B,), jnp.float32),
    )(ou, lu, oi, li, att1_t, att2_t)


def kernel(user_emb, item_emb, att_exp1, att_exp2, edge_weight, edge_index,
           users, items):
    emb0 = jnp.concatenate([user_emb, item_emb], axis=0)
    src = jnp.pad(edge_index[0], (0, _EPAD - _E))
    dst = jnp.pad(edge_index[1], (0, _EPAD - _E))
    w = jnp.pad(edge_weight, (0, _EPAD - _E))

    p = _layer(emb0, src, dst, w)
    emb1, acc1 = _combine(p[0], p[1], emb0)
    q = _layer(emb1, src, dst, w)
    emb2, acc2 = _combine(q[0], q[1], acc1)
    r = _layer(emb2, src, dst, w)
    light = _light(r[0], r[1], acc2)

    ou, lu, oi, li = _gather_batch(user_emb, item_emb, light, users, items)
    return _gamma(ou, lu, oi, li, att_exp1.T, att_exp2.T)
